# Initial kernel scaffold; baseline (speedup 1.0000x reference)
import functools

import jax
import jax.numpy as jnp
from jax import lax
from jax.experimental import pallas as pl
from jax.experimental.pallas import tpu as pltpu
from jax.experimental.pallas import tpu_sc as plsc

_NC = 2
_NS = 16
_NW = _NC * _NS
_CH = 128


def _sc_segment_sum(src2d, dst2d, feat, zeros, n_pad, f, k, interpret=False):
  mesh = plsc.VectorSubcoreMesh(core_axis_name="c", subcore_axis_name="s")
  rpt = n_pad // _NS

  def body(src_hbm, dst_hbm, feat_hbm, zero_hbm, out_hbm,
           acc_sh, sidx, didx, rows, sem):
    c = lax.axis_index("c")
    s = lax.axis_index("s")
    wid = c * _NS + s
    pltpu.sync_copy(zero_hbm.at[pl.ds(s * rpt, rpt)],
                    acc_sh.at[pl.ds(s * rpt, rpt)])
    pltpu.sync_copy(src_hbm.at[pl.ds(wid * k, k)], sidx)
    pltpu.sync_copy(dst_hbm.at[pl.ds(wid * k, k)], didx)
    plsc.subcore_barrier()

    def step(j, carry):
      pltpu.async_copy(feat_hbm.at[sidx.at[j]], rows, sem).wait()
      pltpu.sync_copy(rows, acc_sh.at[didx.at[j]], add=True)
      return carry

    lax.fori_loop(0, k, step, 0)
    plsc.subcore_barrier()
    pltpu.sync_copy(acc_sh.at[pl.ds(s * rpt, rpt)],
                    out_hbm.at[c, pl.ds(s * rpt, rpt)])

  run = pl.kernel(
      body,
      out_type=jax.ShapeDtypeStruct((_NC, n_pad, f), jnp.float32),
      mesh=mesh,
      scratch_types=[
          pltpu.VMEM_SHARED((n_pad, f), jnp.float32),
          pltpu.VMEM((k, _CH), jnp.int32),
          pltpu.VMEM((k, _CH), jnp.int32),
          pltpu.VMEM((_CH, f), jnp.float32),
          pltpu.SemaphoreType.DMA,
      ],
      interpret=interpret,
  )
  return run(src2d, dst2d, feat, zeros)


def _tc1(x, w1n, w1r, b1, bn, interpret=False):
  n, d = x.shape

  def body(x_ref, w1n_ref, w1r_ref, b1_ref, p1_ref, r1_ref):
    xb = x_ref[...]
    p = jnp.dot(xb, w1n_ref[...], preferred_element_type=jnp.float32)
    pad = jnp.concatenate(
        [jnp.ones((bn, 1), jnp.float32), jnp.zeros((bn, 15), jnp.float32)],
        axis=1)
    p1_ref[...] = jnp.concatenate([p, pad], axis=1)
    r1_ref[...] = (jnp.dot(xb, w1r_ref[...], preferred_element_type=jnp.float32)
                   + b1_ref[...])

  return pl.pallas_call(
      body,
      grid=(n // bn,),
      in_specs=[
          pl.BlockSpec((bn, d), lambda i: (i, 0)),
          pl.BlockSpec((d, d), lambda i: (0, 0)),
          pl.BlockSpec((d, d), lambda i: (0, 0)),
          pl.BlockSpec((1, d), lambda i: (0, 0)),
      ],
      out_specs=[
          pl.BlockSpec((bn, d + 16), lambda i: (i, 0)),
          pl.BlockSpec((bn, d), lambda i: (i, 0)),
      ],
      out_shape=[
          jax.ShapeDtypeStruct((n, d + 16), jnp.float32),
          jax.ShapeDtypeStruct((n, d), jnp.float32),
      ],
      interpret=interpret,
  )(x, w1n, w1r, b1.reshape(1, d))


def _tc2(agg1, r1, w2n, w2r, wcp, b2, bcp, bn, interpret=False):
  _, n_pad, f1 = agg1.shape
  n, d = r1.shape

  def body(agg_ref, r1_ref, w2n_ref, w2r_ref, wcp_ref, b2_ref, bcp_ref,
           q_ref, r2_ref, invd_ref):
    agg = agg_ref[0] + agg_ref[1]
    deg = agg[:, d:d + 1]
    invd = 1.0 / jnp.maximum(deg, 1.0)
    h1 = jnp.maximum(agg[:, :d] * invd + r1_ref[...], 0.0)
    w2nc = jnp.dot(w2n_ref[...], wcp_ref[...],
                   preferred_element_type=jnp.float32)
    w2rc = jnp.dot(w2r_ref[...], wcp_ref[...],
                   preferred_element_type=jnp.float32)
    bc2 = jnp.dot(b2_ref[...], wcp_ref[...],
                  preferred_element_type=jnp.float32) + bcp_ref[...]
    q_ref[...] = jnp.dot(h1, w2nc, preferred_element_type=jnp.float32)
    r2_ref[...] = jnp.dot(h1, w2rc, preferred_element_type=jnp.float32) + bc2
    invd_ref[...] = invd

  return pl.pallas_call(
      body,
      grid=(n // bn,),
      in_specs=[
          pl.BlockSpec((2, bn, f1), lambda i: (0, i, 0)),
          pl.BlockSpec((bn, d), lambda i: (i, 0)),
          pl.BlockSpec((d, d), lambda i: (0, 0)),
          pl.BlockSpec((d, d), lambda i: (0, 0)),
          pl.BlockSpec((d, 16), lambda i: (0, 0)),
          pl.BlockSpec((1, d), lambda i: (0, 0)),
          pl.BlockSpec((1, 16), lambda i: (0, 0)),
      ],
      out_specs=[
          pl.BlockSpec((bn, 16), lambda i: (i, 0)),
          pl.BlockSpec((bn, 16), lambda i: (i, 0)),
          pl.BlockSpec((bn, 1), lambda i: (i, 0)),
      ],
      out_shape=[
          jax.ShapeDtypeStruct((n, 16), jnp.float32),
          jax.ShapeDtypeStruct((n, 16), jnp.float32),
          jax.ShapeDtypeStruct((n, 1), jnp.float32),
      ],
      interpret=interpret,
  )(agg1, r1, w2n, w2r, wcp, b2.reshape(1, d), bcp.reshape(1, 16))


def _tc3(agg2, r2, invd, bn, interpret=False):
  _, n_pad, f2 = agg2.shape
  n = r2.shape[0]

  def body(agg_ref, r2_ref, invd_ref, out_ref):
    out_ref[...] = (agg_ref[0] + agg_ref[1]) * invd_ref[...] + r2_ref[...]

  return pl.pallas_call(
      body,
      grid=(n // bn,),
      in_specs=[
          pl.BlockSpec((2, bn, f2), lambda i: (0, i, 0)),
          pl.BlockSpec((bn, 16), lambda i: (i, 0)),
          pl.BlockSpec((bn, 1), lambda i: (i, 0)),
      ],
      out_specs=pl.BlockSpec((bn, 16), lambda i: (i, 0)),
      out_shape=jax.ShapeDtypeStruct((n, 16), jnp.float32),
      interpret=interpret,
  )(agg2, r2, invd)


def _impl(x, edge_index, w1_neigh, w1_root, b1, w2_neigh, w2_root, b2, wc, bc,
          interpret=False):
  n, d = x.shape
  e = edge_index.shape[1]
  c_out = wc.shape[1]

  chunk = _CH * _NW
  e_pad = -(-e // chunk) * chunk
  k = e_pad // chunk
  n_pad = -(-(n + 1) // _NS) * _NS
  f1 = d + 16

  src = jnp.concatenate(
      [edge_index[0], jnp.zeros((e_pad - e,), jnp.int32)]).reshape(_NW * k, _CH)
  dst = jnp.concatenate(
      [edge_index[1], jnp.full((e_pad - e,), n, jnp.int32)]).reshape(_NW * k, _CH)
  wcp = jnp.pad(wc, ((0, 0), (0, 16 - c_out)))
  bcp = jnp.pad(bc, (0, 16 - c_out))

  bn = 400 if n % 400 == 0 else 8 * (n // 8)

  p1aug, r1 = _tc1(x, w1_neigh, w1_root, b1, bn, interpret)
  agg1 = _sc_segment_sum(src, dst, p1aug, jnp.zeros((n_pad, f1), jnp.float32),
                         n_pad, f1, k, interpret)
  q, r2, invd = _tc2(agg1, r1, w2_neigh, w2_root, wcp, b2, bcp, bn, interpret)
  agg2 = _sc_segment_sum(src, dst, q, jnp.zeros((n_pad, 16), jnp.float32),
                         n_pad, 16, k, interpret)
  logits16 = _tc3(agg2, r2, invd, bn, interpret)
  return logits16[:, :c_out]


def kernel(x, edge_index, w1_neigh, w1_root, b1, w2_neigh, w2_root, b2, wc, bc):
  return _impl(x, edge_index, w1_neigh, w1_root, b1,
               w2_neigh, w2_root, b2, wc, bc)


# speedup vs baseline: 6.6660x; 6.6660x over previous
import functools

import jax
import jax.numpy as jnp
from jax import lax
from jax.experimental import pallas as pl
from jax.experimental.pallas import tpu as pltpu
from jax.experimental.pallas import tpu_sc as plsc

_NC = 2
_NS = 16
_NW = _NC * _NS
_CH = 128


def _sc_segment_sum(src2d, dst2d, feat, zeros, n_pad, f, k, interpret=False):
  mesh = plsc.VectorSubcoreMesh(core_axis_name="c", subcore_axis_name="s",
                                num_cores=_NC, num_subcores=_NS)
  rpt = n_pad // _NS

  def body(src_hbm, dst_hbm, feat_hbm, zero_hbm, out_hbm,
           acc_sh, sidx, didx, rows, sem):
    c = lax.axis_index("c")
    s = lax.axis_index("s")
    wid = c * _NS + s
    pltpu.sync_copy(zero_hbm.at[pl.ds(s * rpt, rpt)],
                    acc_sh.at[pl.ds(s * rpt, rpt)])
    pltpu.sync_copy(src_hbm.at[pl.ds(wid * k, k)], sidx)
    pltpu.sync_copy(dst_hbm.at[pl.ds(wid * k, k)], didx)
    plsc.subcore_barrier()

    def step(j, carry):
      pltpu.async_copy(feat_hbm.at[sidx.at[j]], rows, sem).wait()
      pltpu.sync_copy(rows, acc_sh.at[didx.at[j]], add=True)
      return carry

    lax.fori_loop(0, k, step, 0)
    plsc.subcore_barrier()
    pltpu.sync_copy(acc_sh.at[pl.ds(s * rpt, rpt)],
                    out_hbm.at[c, pl.ds(s * rpt, rpt)])

  run = pl.kernel(
      body,
      out_type=jax.ShapeDtypeStruct((_NC, n_pad, f), jnp.float32),
      mesh=mesh,
      scratch_types=[
          pltpu.VMEM_SHARED((n_pad, f), jnp.float32),
          pltpu.VMEM((k, _CH), jnp.int32),
          pltpu.VMEM((k, _CH), jnp.int32),
          pltpu.VMEM((_CH, f), jnp.float32),
          pltpu.SemaphoreType.DMA,
      ],
      compiler_params=pltpu.CompilerParams(use_tc_tiling_on_sc=False),
      interpret=interpret,
  )
  return run(src2d, dst2d, feat, zeros)


def _tc1(x, w1n, w1r, b1, bn, interpret=False):
  n, d = x.shape

  def body(x_ref, w1n_ref, w1r_ref, b1_ref, p1_ref, r1_ref):
    xb = x_ref[...]
    p = jnp.dot(xb, w1n_ref[...], preferred_element_type=jnp.float32)
    pad = jnp.concatenate(
        [jnp.ones((bn, 1), jnp.float32), jnp.zeros((bn, 15), jnp.float32)],
        axis=1)
    p1_ref[...] = jnp.concatenate([p, pad], axis=1)
    r1_ref[...] = (jnp.dot(xb, w1r_ref[...], preferred_element_type=jnp.float32)
                   + b1_ref[...])

  return pl.pallas_call(
      body,
      grid=(n // bn,),
      in_specs=[
          pl.BlockSpec((bn, d), lambda i: (i, 0)),
          pl.BlockSpec((d, d), lambda i: (0, 0)),
          pl.BlockSpec((d, d), lambda i: (0, 0)),
          pl.BlockSpec((1, d), lambda i: (0, 0)),
      ],
      out_specs=[
          pl.BlockSpec((bn, d + 16), lambda i: (i, 0)),
          pl.BlockSpec((bn, d), lambda i: (i, 0)),
      ],
      out_shape=[
          jax.ShapeDtypeStruct((n, d + 16), jnp.float32),
          jax.ShapeDtypeStruct((n, d), jnp.float32),
      ],
      interpret=interpret,
  )(x, w1n, w1r, b1.reshape(1, d))


def _tc2(agg1, r1, w2n, w2r, wcp, b2, bcp, bn, interpret=False):
  _, n_pad, f1 = agg1.shape
  n, d = r1.shape

  def body(agg_ref, r1_ref, w2n_ref, w2r_ref, wcp_ref, b2_ref, bcp_ref,
           q_ref, r2_ref, invd_ref):
    agg = agg_ref[0] + agg_ref[1]
    deg = agg[:, d:d + 1]
    invd = 1.0 / jnp.maximum(deg, 1.0)
    h1 = jnp.maximum(agg[:, :d] * invd + r1_ref[...], 0.0)
    w2nc = jnp.dot(w2n_ref[...], wcp_ref[...],
                   preferred_element_type=jnp.float32)
    w2rc = jnp.dot(w2r_ref[...], wcp_ref[...],
                   preferred_element_type=jnp.float32)
    bc2 = jnp.dot(b2_ref[...], wcp_ref[...],
                  preferred_element_type=jnp.float32) + bcp_ref[...]
    q_ref[...] = jnp.dot(h1, w2nc, preferred_element_type=jnp.float32)
    r2_ref[...] = jnp.dot(h1, w2rc, preferred_element_type=jnp.float32) + bc2
    invd_ref[...] = invd

  return pl.pallas_call(
      body,
      grid=(n // bn,),
      in_specs=[
          pl.BlockSpec((2, bn, f1), lambda i: (0, i, 0)),
          pl.BlockSpec((bn, d), lambda i: (i, 0)),
          pl.BlockSpec((d, d), lambda i: (0, 0)),
          pl.BlockSpec((d, d), lambda i: (0, 0)),
          pl.BlockSpec((d, 16), lambda i: (0, 0)),
          pl.BlockSpec((1, d), lambda i: (0, 0)),
          pl.BlockSpec((1, 16), lambda i: (0, 0)),
      ],
      out_specs=[
          pl.BlockSpec((bn, 16), lambda i: (i, 0)),
          pl.BlockSpec((bn, 16), lambda i: (i, 0)),
          pl.BlockSpec((bn, 1), lambda i: (i, 0)),
      ],
      out_shape=[
          jax.ShapeDtypeStruct((n, 16), jnp.float32),
          jax.ShapeDtypeStruct((n, 16), jnp.float32),
          jax.ShapeDtypeStruct((n, 1), jnp.float32),
      ],
      interpret=interpret,
  )(agg1, r1, w2n, w2r, wcp, b2.reshape(1, d), bcp.reshape(1, 16))


def _tc3(agg2, r2, invd, bn, interpret=False):
  _, n_pad, f2 = agg2.shape
  n = r2.shape[0]

  def body(agg_ref, r2_ref, invd_ref, out_ref):
    out_ref[...] = (agg_ref[0] + agg_ref[1]) * invd_ref[...] + r2_ref[...]

  return pl.pallas_call(
      body,
      grid=(n // bn,),
      in_specs=[
          pl.BlockSpec((2, bn, f2), lambda i: (0, i, 0)),
          pl.BlockSpec((bn, 16), lambda i: (i, 0)),
          pl.BlockSpec((bn, 1), lambda i: (i, 0)),
      ],
      out_specs=pl.BlockSpec((bn, 16), lambda i: (i, 0)),
      out_shape=jax.ShapeDtypeStruct((n, 16), jnp.float32),
      interpret=interpret,
  )(agg2, r2, invd)


def _impl(x, edge_index, w1_neigh, w1_root, b1, w2_neigh, w2_root, b2, wc, bc,
          interpret=False):
  n, d = x.shape
  e = edge_index.shape[1]
  c_out = wc.shape[1]

  chunk = _CH * _NW
  e_pad = -(-e // chunk) * chunk
  k = e_pad // chunk
  n_pad = -(-(n + 1) // (_NS * 8)) * (_NS * 8)
  f1 = d + 16

  src = jnp.concatenate(
      [edge_index[0], jnp.zeros((e_pad - e,), jnp.int32)]).reshape(_NW * k, _CH)
  dst = jnp.concatenate(
      [edge_index[1], jnp.full((e_pad - e,), n, jnp.int32)]).reshape(_NW * k, _CH)
  wcp = jnp.pad(wc, ((0, 0), (0, 16 - c_out)))
  bcp = jnp.pad(bc, (0, 16 - c_out))

  bn = 400 if n % 400 == 0 else 8 * (n // 8)

  p1aug, r1 = _tc1(x, w1_neigh, w1_root, b1, bn, interpret)
  agg1 = _sc_segment_sum(src, dst, p1aug, jnp.zeros((n_pad, f1), jnp.float32),
                         n_pad, f1, k, interpret)
  q, r2, invd = _tc2(agg1, r1, w2_neigh, w2_root, wcp, b2, bcp, bn, interpret)
  agg2 = _sc_segment_sum(src, dst, q, jnp.zeros((n_pad, 16), jnp.float32),
                         n_pad, 16, k, interpret)
  logits16 = _tc3(agg2, r2, invd, bn, interpret)
  return logits16[:, :c_out]


def kernel(x, edge_index, w1_neigh, w1_root, b1, w2_neigh, w2_root, b2, wc, bc):
  return _impl(x, edge_index, w1_neigh, w1_root, b1,
               w2_neigh, w2_root, b2, wc, bc)
